# Initial kernel scaffold; baseline (speedup 1.0000x reference)
import jax
import jax.numpy as jnp
from jax.experimental import pallas as pl


def kernel(vfeat, efeat, src_nodes, dst_edges, src_edges, dst_nodes, Wvtx, bvtx, Wqe, bqe, Wkv, bkv, Wvv, bvv, Wqv, bqv, Wke, bke, Wve, bve):
    raise NotImplementedError("write your pallas kernel here")


# speedup vs baseline: 5.7764x; 5.7764x over previous
import dataclasses
import functools

import numpy as np
import jax
import jax.numpy as jnp
from jax import lax
from jax.experimental import pallas as pl
from jax.experimental.pallas import tpu as pltpu
from jax.experimental.pallas import tpu_sc as plsc

_NC = 2
_NS = 16
_NW = _NC * _NS
_L = 16
_CHUNK = 80


def _make_sc_edge_pass(n_seg, n_edges):
    pad = (-n_seg) % (_NS * 8)
    P = n_seg + pad
    assert P % 128 == 0
    rows_per_tile = P // _NS
    epw = n_edges // _NW
    n_iters = epw // _CHUNK
    assert n_edges == epw * _NW and epw == n_iters * _CHUNK
    mesh = plsc.VectorSubcoreMesh(core_axis_name="c", subcore_axis_name="s")
    cp = pltpu.CompilerParams()
    if "needs_layout_passes" in pltpu.CompilerParams.__dataclass_fields__:
        cp = dataclasses.replace(cp, needs_layout_passes=False)

    @functools.partial(
        pl.kernel,
        compiler_params=cp,
        out_type=jax.ShapeDtypeStruct((_NC, P, 128), jnp.float32),
        mesh=mesh,
        scratch_types=[
            pltpu.VMEM((_CHUNK,), jnp.int32),
            pltpu.VMEM((_CHUNK,), jnp.int32),
            pltpu.VMEM((_CHUNK,), jnp.int32),
            pltpu.VMEM((_CHUNK, 128), jnp.float32),
            pltpu.VMEM((_CHUNK, 128), jnp.float32),
            pltpu.VMEM((_CHUNK,), jnp.float32),
            pltpu.VMEM((_CHUNK, 128), jnp.float32),
            pltpu.VMEM((8, 128), jnp.float32),
            pltpu.VMEM_SHARED((P, 128), jnp.float32),
        ],
    )
    def sc_pass(kv_hbm, q_hbm, src_hbm, dst_hbm, out_hbm,
                src_v, dst_v, dsth_v, kv_rows, q_rows, w_v, out_rows,
                zbuf, acc):
        c = lax.axis_index("c")
        s = lax.axis_index("s")
        wid = c * _NS + s
        zeros16 = jnp.zeros((_L,), jnp.float32)

        @pl.loop(0, 8)
        def _(r):
            for j in range(8):
                zbuf[r, pl.ds(j * _L, _L)] = zeros16

        @pl.loop(0, rows_per_tile // 8)
        def _(r):
            pltpu.sync_copy(zbuf,
                            acc.at[pl.ds(s * rows_per_tile + r * 8, 8)])

        @pl.loop(0, _CHUNK)
        def _(e):
            for j in range(5, 8):
                out_rows[e, pl.ds(j * _L, _L)] = zeros16

        plsc.subcore_barrier()

        lane = lax.iota(jnp.int32, _L)
        one0 = jnp.where(lane == 0, jnp.float32(1.0), jnp.float32(0.0))
        scale = jnp.float32(1.0 / np.sqrt(64.0))
        base = wid * epw

        @pl.loop(0, n_iters)
        def _(t):
            e0 = base + t * _CHUNK
            pltpu.sync_copy(src_hbm.at[pl.ds(e0, _CHUNK)], src_v)
            pltpu.sync_copy(dst_hbm.at[pl.ds(e0, _CHUNK)], dst_v)

            for g in range(_CHUNK // _L):
                d16 = dst_v[pl.ds(g * _L, _L)]
                dsth_v[pl.ds(g * _L, _L)] = lax.shift_right_logical(d16, 1)

            pltpu.sync_copy(kv_hbm.at[src_v], kv_rows)
            pltpu.sync_copy(q_hbm.at[dsth_v], q_rows)

            for g in range(_CHUNK // _L):
                row_idx = lane + g * _L
                qcol0 = (dst_v[pl.ds(g * _L, _L)] & 1) * 64

                def dot_step(d0, acc_v, row_idx=row_idx, qcol0=qcol0):
                    for i in range(16):
                        d = d0 * 16 + i
                        col = jnp.full((_L,), d, jnp.int32)
                        kv_d = plsc.load_gather(kv_rows, [row_idx, col])
                        q_d = plsc.load_gather(q_rows, [row_idx, qcol0 + d])
                        acc_v = acc_v + kv_d * q_d
                    return acc_v

                sdot = lax.fori_loop(0, 4, dot_step, zeros16)
                lrelu = jnp.where(sdot >= 0, sdot, sdot * jnp.float32(0.01))
                w_v[pl.ds(g * _L, _L)] = jnp.exp(lrelu * scale)

            @pl.loop(0, _CHUNK)
            def _(e):
                wv = plsc.load_gather(w_v, [jnp.full((_L,), e, jnp.int32)])
                for j in range(4):
                    out_rows[e, pl.ds(j * _L, _L)] = (
                        wv * kv_rows[e, pl.ds(64 + j * _L, _L)])
                out_rows[e, pl.ds(64, _L)] = wv * one0

            pltpu.sync_copy(out_rows, acc.at[dst_v], add=True)

        plsc.subcore_barrier()
        r0 = s * rows_per_tile
        pltpu.sync_copy(acc.at[pl.ds(r0, rows_per_tile)],
                        out_hbm.at[c, pl.ds(r0, rows_per_tile)])

    return sc_pass


def _tc_dense1(vfeat, efeat, Wvtx, bvtx, Wc1, bc1, Wqe, bqe, Wqv, bqv):
    n_v = vfeat.shape[0]
    n_e = efeat.shape[0]

    def body(v_ref, e_ref, wv_ref, bv_ref, wc_ref, bc_ref, wq_ref, bq_ref,
             wq2_ref, bq2_ref, fv_ref, kv1_ref, q1_ref, q2_ref):
        fv = jnp.dot(v_ref[...], wv_ref[...],
                     preferred_element_type=jnp.float32) + bv_ref[...]
        fv_ref[...] = fv
        kv1_ref[...] = jnp.dot(fv, wc_ref[...],
                               preferred_element_type=jnp.float32) + bc_ref[...]
        q1_ref[...] = jnp.dot(e_ref[...], wq_ref[...],
                              preferred_element_type=jnp.float32) + bq_ref[...]
        q2_ref[...] = jnp.dot(fv, wq2_ref[...],
                              preferred_element_type=jnp.float32) + bq2_ref[...]

    return pl.pallas_call(
        body,
        out_shape=[
            jax.ShapeDtypeStruct((n_v, 64), jnp.float32),
            jax.ShapeDtypeStruct((n_v, 128), jnp.float32),
            jax.ShapeDtypeStruct((n_e, 64), jnp.float32),
            jax.ShapeDtypeStruct((n_v, 64), jnp.float32),
        ],
    )(vfeat, efeat, Wvtx, bvtx, Wc1, bc1, Wqe, bqe, Wqv, bqv)


def _tc_mid(part1, Wc2, bc2, n_seg):
    def body(p_ref, w_ref, b_ref, fe_ref, kv2_ref):
        p = p_ref[0] + p_ref[1]
        h = p[:n_seg, :64]
        den = p[:n_seg, 64:65]
        den = jnp.where(den == 0.0, jnp.float32(1.0), den)
        fe = jnp.maximum(h / den, 0.0)
        fe_ref[...] = fe
        kv2_ref[...] = jnp.dot(fe, w_ref[...],
                               preferred_element_type=jnp.float32) + b_ref[...]

    return pl.pallas_call(
        body,
        out_shape=[
            jax.ShapeDtypeStruct((n_seg, 64), jnp.float32),
            jax.ShapeDtypeStruct((n_seg, 128), jnp.float32),
        ],
    )(part1, Wc2, bc2)


def _tc_fin(part2, n_seg):
    def body(p_ref, o_ref):
        p = p_ref[0] + p_ref[1]
        h = p[:n_seg, :64]
        den = p[:n_seg, 64:65]
        den = jnp.where(den == 0.0, jnp.float32(1.0), den)
        o_ref[...] = jnp.maximum(h / den, 0.0)

    return pl.pallas_call(
        body,
        out_shape=jax.ShapeDtypeStruct((n_seg, 64), jnp.float32),
    )(part2)


def kernel(vfeat, efeat, src_nodes, dst_edges, src_edges, dst_nodes,
           Wvtx, bvtx, Wqe, bqe, Wkv, bkv, Wvv, bvv, Wqv, bqv,
           Wke, bke, Wve, bve):
    n_v = vfeat.shape[0]
    n_e = efeat.shape[0]
    n_edges = src_nodes.shape[0]
    src_n = src_nodes.astype(jnp.int32)
    dst_e = dst_edges.astype(jnp.int32)
    src_e = src_edges.astype(jnp.int32)
    dst_n = dst_nodes.astype(jnp.int32)

    Wc1 = jnp.concatenate([Wkv, Wvv], axis=1)
    bc1 = jnp.concatenate([bkv, bvv])[None, :]
    Wc2 = jnp.concatenate([Wke, Wve], axis=1)
    bc2 = jnp.concatenate([bke, bve])[None, :]

    fv, kv1, q1, q2 = _tc_dense1(vfeat, efeat, Wvtx, bvtx[None, :],
                                 Wc1, bc1, Wqe, bqe[None, :], Wqv, bqv[None, :])
    q1p = jnp.reshape(q1, (n_e // 2, 128))
    q2p = jnp.reshape(q2, (n_v // 2, 128))
    part1 = _make_sc_edge_pass(n_e, n_edges)(kv1, q1p, src_n, dst_e)
    fe, kv2 = _tc_mid(part1, Wc2, bc2, n_e)
    part2 = _make_sc_edge_pass(n_v, n_edges)(kv2, q2p, src_e, dst_n)
    fv_out = _tc_fin(part2, n_v)
    return fv_out, fe
